# baseline (speedup 1.0000x reference)
import functools

import jax
import jax.numpy as jnp
from jax import lax
from jax.experimental import pallas as pl
from jax.experimental.pallas import tpu as pltpu
from jax.experimental.pallas import tpu_sc as plsc

N_FULL = 100000
N_SUB = 10000
E = 160000
F_IN = 256
HID = 256
NUM_CLASSES = 41

NC = 2
NS = 16
N_PAD = 10240
ROWS_W = N_PAD // (NC * NS)
LAB_PAD = 128

CHUNK = 64
NCHUNK = 80
E_PAD = NC * NS * CHUNK * NCHUNK
WIN = 2560
NWIN = N_PAD // WIN
ACC_R = WIN + 4
ROWS_P = WIN // NS

_mesh = functools.partial(
    plsc.VectorSubcoreMesh, core_axis_name="c", subcore_axis_name="s",
    num_cores=NC, num_subcores=NS)

_BCAST_DNUMS = lax.GatherDimensionNumbers(
    offset_dims=(), collapsed_slice_dims=(0,), start_index_map=(0,))


def _lane_bcast(v16, lane):
    idx = jnp.full((16, 1), lane, jnp.int32)
    return lax.gather(v16, idx, _BCAST_DNUMS, (1,),
                      mode=lax.GatherScatterMode.PROMISE_IN_BOUNDS)


def _gather_body(idx_hbm, feat_hbm, lab_hbm, feat_out, lab_out,
                 idx_v, feat_v, lab_v, sem):
    wid = lax.axis_index("s") * NC + lax.axis_index("c")
    base = wid * ROWS_W
    pltpu.sync_copy(idx_hbm.at[pl.ds(base, ROWS_W)], idx_v)
    pltpu.async_copy(feat_hbm.at[idx_v], feat_v, sem).wait()
    pltpu.sync_copy(feat_v, feat_out.at[pl.ds(base, ROWS_W)])
    pltpu.async_copy(lab_hbm.at[idx_v], lab_v, sem).wait()
    pltpu.sync_copy(lab_v, lab_out.at[pl.ds(base, ROWS_W)])


def _sc_gather(idx_pad, feat_full, lab_pad_full):
    return pl.kernel(
        _gather_body,
        out_type=(jax.ShapeDtypeStruct((N_PAD, F_IN), jnp.float32),
                  jax.ShapeDtypeStruct((N_PAD, LAB_PAD), jnp.float32)),
        name="subg_gather",
        mesh=_mesh(),
        scratch_types=(
            pltpu.VMEM((ROWS_W,), jnp.int32),
            pltpu.VMEM((ROWS_W, F_IN), jnp.float32),
            pltpu.VMEM((ROWS_W, LAB_PAD), jnp.float32),
            pltpu.SemaphoreType.DMA,
        ),
    )(idx_pad, feat_full, lab_pad_full)


def _spmm_body(u_hbm, col_hbm, row_hbm, val_hbm, out_hbm,
               col_v, row_v, val_v, row_adj, gath0, gath1, scaled,
               acc, g0, g1, s0):
    cid = lax.axis_index("c")
    sid = lax.axis_index("s")

    ebase = cid * (NS * NCHUNK) + sid * NCHUNK
    pltpu.sync_copy(col_hbm.at[pl.ds(ebase, NCHUNK)],
                    col_v.at[pl.ds(0, NCHUNK)])
    pltpu.sync_copy(row_hbm.at[pl.ds(ebase, NCHUNK)],
                    row_v.at[pl.ds(0, NCHUNK)])
    pltpu.sync_copy(val_hbm.at[pl.ds(ebase, NCHUNK)],
                    val_v.at[pl.ds(0, NCHUNK)])
    zv16 = jnp.zeros((16,), jnp.float32)
    zi16 = jnp.zeros((16,), jnp.int32)
    for d in range(2):
        for j in range(CHUNK // 16):
            col_v[NCHUNK + d, pl.ds(j * 16, 16)] = zi16
            row_v[NCHUNK + d, pl.ds(j * 16, 16)] = zi16
            val_v[NCHUNK + d, pl.ds(j * 16, 16)] = zv16


    def _scale(gb, k, half):
        def _grp(b, _):
            val16 = val_v[k, pl.ds(pl.multiple_of(b * 16, 16), 16)]
            for lane in range(16):
                vb = _lane_bcast(val16, lane)
                j = b * 16 + lane
                jo = half * CHUNK + j
                for g in range(8):
                    w = gb[j, pl.ds(g * 16, 16)]
                    a = lax.bitcast_convert_type(w << 16, jnp.float32)
                    b2 = lax.bitcast_convert_type(
                        w & jnp.int32(-65536), jnp.float32)
                    scaled[jo, 0, pl.ds(g * 16, 16)] = a * vb
                    scaled[jo, 1, pl.ds(g * 16, 16)] = b2 * vb
            return 0
        lax.fori_loop(0, CHUNK // 16, _grp, 0)

    def _pass(p, _):
        lo = pl.multiple_of(p * WIN, WIN)
        def _zs(i, _):
            for d in range(2):
                for j in range(8):
                    scaled[i, d, pl.ds(j * 16, 16)] = zv16
            return 0
        lax.fori_loop(0, 128, _zs, 0)
        base = sid * ROWS_P
        pltpu.sync_copy(scaled, acc.at[pl.ds(base, 128)])
        pltpu.sync_copy(scaled.at[pl.ds(0, 32)],
                        acc.at[pl.ds(base + 128, 32)])
        plsc.subcore_barrier()

        def _radj(m, _):
            for j in range(2 * CHUNK // 16):
                r = row_v[2 * m + j // 4, pl.ds((j % 4) * 16, 16)] - lo
                ok = (r >= 0) & (r < WIN)
                row_adj[m, pl.ds(j * 16, 16)] = jnp.where(ok, r, WIN)
            return 0
        lax.fori_loop(0, (NCHUNK + 2) // 2, _radj, 0)

        def _scan(k, carry):
            below, above = carry
            cmax = row_v[k, pl.ds(CHUNK - 16, 16)][15]
            cmin = row_v[k, pl.ds(0, 16)][0]
            below = below + jnp.where(cmax < lo, 1, 0)
            above = above + jnp.where(cmin >= lo + WIN, 1, 0)
            return (below, above)
        below, above = lax.fori_loop(0, NCHUNK, _scan, (0, 0))
        k_lo = below
        k_hi = NCHUNK - above

        m_lo = k_lo // 2
        m_hi = (k_hi + 1) // 2
        pltpu.async_copy(u_hbm.at[col_v.at[2 * m_lo]], gath0, g0)
        pltpu.async_copy(u_hbm.at[col_v.at[2 * m_lo + 1]], gath1, g1)

        def _pair(i, _):
            m = m_lo + i
            pltpu.make_async_copy(
                u_hbm.at[col_v.at[2 * m]], gath0, g0).wait()
            _scale(gath0, 2 * m, 0)
            pltpu.make_async_copy(
                u_hbm.at[col_v.at[2 * m + 1]], gath1, g1).wait()
            _scale(gath1, 2 * m + 1, 1)
            pltpu.async_copy(scaled, acc.at[row_adj.at[m]], s0, add=True)
            pltpu.make_async_copy(scaled, acc.at[row_adj.at[m]], s0).wait()
            mp = jnp.minimum(m + 1, NCHUNK // 2)
            pltpu.async_copy(u_hbm.at[col_v.at[2 * mp]], gath0, g0)
            pltpu.async_copy(u_hbm.at[col_v.at[2 * mp + 1]], gath1, g1)
            return 0
        lax.fori_loop(0, m_hi - m_lo, _pair, 0)

        pltpu.make_async_copy(u_hbm.at[col_v.at[0]], gath0, g0).wait()
        pltpu.make_async_copy(u_hbm.at[col_v.at[0]], gath1, g1).wait()

        plsc.subcore_barrier()
        off = pl.multiple_of(cid * N_PAD + lo + sid * ROWS_P, 8)
        pltpu.sync_copy(acc.at[pl.ds(sid * ROWS_P, ROWS_P)],
                        out_hbm.at[pl.ds(off, ROWS_P)])
        plsc.subcore_barrier()
        return 0
    lax.fori_loop(0, NWIN, _pass, 0)


def _sc_spmm(u_packed, col2d, row2d, val2d):
    return pl.kernel(
        _spmm_body,
        out_type=jax.ShapeDtypeStruct((2 * N_PAD, 2, 128), jnp.float32),
        name="spmm",
        mesh=_mesh(),
        scratch_types=(
            pltpu.VMEM((NCHUNK + 2, CHUNK), jnp.int32),
            pltpu.VMEM((NCHUNK + 2, CHUNK), jnp.int32),
            pltpu.VMEM((NCHUNK + 2, CHUNK), jnp.float32),
            pltpu.VMEM(((NCHUNK + 2) // 2, 2 * CHUNK), jnp.int32),
            pltpu.VMEM((CHUNK, 128), jnp.int32),
            pltpu.VMEM((CHUNK, 128), jnp.int32),
            pltpu.VMEM((2 * CHUNK, 2, 128), jnp.float32),
            pltpu.VMEM_SHARED((ACC_R, 2, 128), jnp.float32),
            pltpu.SemaphoreType.DMA,
            pltpu.SemaphoreType.DMA,
            pltpu.SemaphoreType.DMA,
        ),
    )(u_packed, col2d, row2d, val2d)


def _bf16_pack(u):
    lo = lax.bitcast_convert_type(
        u[:, :128].astype(jnp.bfloat16), jnp.uint16).astype(jnp.uint32)
    hi = lax.bitcast_convert_type(
        u[:, 128:].astype(jnp.bfloat16), jnp.uint16).astype(jnp.uint32)
    return lax.bitcast_convert_type((hi << 16) | lo, jnp.int32)


def _k1_body(x_ref, ws_ref, wh_ref, bs_ref, t1_ref, u_ref):
    x = x_ref[...]
    t1_ref[...] = jnp.maximum(
        jnp.dot(x, ws_ref[...], preferred_element_type=jnp.float32)
        + bs_ref[...], 0.0)
    u_ref[...] = _bf16_pack(
        jnp.dot(x, wh_ref[...], preferred_element_type=jnp.float32))


def _tc_layer1(feat, W1s, W1h, b1s):
    nb = N_PAD // 1024
    return pl.pallas_call(
        _k1_body,
        grid=(nb,),
        in_specs=[
            pl.BlockSpec((1024, F_IN), lambda i: (i, 0)),
            pl.BlockSpec((F_IN, HID), lambda i: (0, 0)),
            pl.BlockSpec((F_IN, HID), lambda i: (0, 0)),
            pl.BlockSpec((1, HID), lambda i: (0, 0)),
        ],
        out_specs=[
            pl.BlockSpec((1024, HID), lambda i: (i, 0)),
            pl.BlockSpec((1024, 128), lambda i: (i, 0)),
        ],
        out_shape=[jax.ShapeDtypeStruct((N_PAD, HID), jnp.float32),
                   jax.ShapeDtypeStruct((N_PAD, 128), jnp.int32)],
    )(feat, W1s, W1h, b1s.reshape(1, HID))


def _k2_body(t1_ref, sa_ref, sb_ref, bh_ref,
             w2s_ref, w2h_ref, b2s_ref, t2_ref, u2_ref):
    t1 = t1_ref[...]
    p1 = jnp.maximum(sa_ref[...] + sb_ref[...] + bh_ref[...], 0.0)
    w2s = w2s_ref[...]
    w2h = w2h_ref[...]
    t2_ref[...] = jnp.maximum(
        jnp.dot(t1, w2s[:HID], preferred_element_type=jnp.float32)
        + jnp.dot(p1, w2s[HID:], preferred_element_type=jnp.float32)
        + b2s_ref[...], 0.0)
    u2_ref[...] = _bf16_pack(
        jnp.dot(t1, w2h[:HID], preferred_element_type=jnp.float32)
        + jnp.dot(p1, w2h[HID:], preferred_element_type=jnp.float32))


def _tc_layer2(t1, s1, b1h, W2s, W2h, b2s):
    nb = N_PAD // 1024
    call = pl.pallas_call(
        _k2_body,
        grid=(nb,),
        in_specs=[
            pl.BlockSpec((1024, HID), lambda i: (i, 0)),
            pl.BlockSpec((1024, HID), lambda i: (i, 0)),
            pl.BlockSpec((1024, HID), lambda i: (nb + i, 0)),
            pl.BlockSpec((1, HID), lambda i: (0, 0)),
            pl.BlockSpec((2 * HID, HID), lambda i: (0, 0)),
            pl.BlockSpec((2 * HID, HID), lambda i: (0, 0)),
            pl.BlockSpec((1, HID), lambda i: (0, 0)),
        ],
        out_specs=[
            pl.BlockSpec((1024, HID), lambda i: (i, 0)),
            pl.BlockSpec((1024, 128), lambda i: (i, 0)),
        ],
        out_shape=[jax.ShapeDtypeStruct((N_PAD, HID), jnp.float32),
                   jax.ShapeDtypeStruct((N_PAD, 128), jnp.int32)],
    )
    return call(t1, s1, s1, b1h.reshape(1, HID), W2s, W2h,
                b2s.reshape(1, HID))


def _k3_body(t2_ref, sa_ref, sb_ref, bh_ref,
             wc_ref, bc_ref, lab_ref, pred_ref, conv_ref):
    t2 = t2_ref[...]
    p2 = jnp.maximum(sa_ref[...] + sb_ref[...] + bh_ref[...], 0.0)
    wc = wc_ref[...]
    z = (jnp.dot(t2, wc[:HID], preferred_element_type=jnp.float32)
         + jnp.dot(p2, wc[HID:], preferred_element_type=jnp.float32))
    nsq = (jnp.sum(t2 * t2, axis=1, keepdims=True)
           + jnp.sum(p2 * p2, axis=1, keepdims=True))
    n = jnp.maximum(jnp.sqrt(nsq), 1e-12)
    pred_ref[...] = z / n + bc_ref[...]
    lab = lab_ref[...][:, :NUM_CLASSES]
    m = jnp.max(lab, axis=1, keepdims=True)
    ii = lax.broadcasted_iota(jnp.int32, lab.shape, 1)
    conv_ref[...] = jnp.min(
        jnp.where(lab == m, ii, NUM_CLASSES), axis=1, keepdims=True)


def _tc_final(t2, s2, b2h, Wc, bc, lab_pad):
    nb = N_PAD // 1024
    call = pl.pallas_call(
        _k3_body,
        grid=(nb,),
        in_specs=[
            pl.BlockSpec((1024, HID), lambda i: (i, 0)),
            pl.BlockSpec((1024, HID), lambda i: (i, 0)),
            pl.BlockSpec((1024, HID), lambda i: (nb + i, 0)),
            pl.BlockSpec((1, HID), lambda i: (0, 0)),
            pl.BlockSpec((2 * HID, NUM_CLASSES), lambda i: (0, 0)),
            pl.BlockSpec((1, NUM_CLASSES), lambda i: (0, 0)),
            pl.BlockSpec((1024, LAB_PAD), lambda i: (i, 0)),
        ],
        out_specs=[
            pl.BlockSpec((1024, NUM_CLASSES), lambda i: (i, 0)),
            pl.BlockSpec((1024, 1), lambda i: (i, 0)),
        ],
        out_shape=[jax.ShapeDtypeStruct((N_PAD, NUM_CLASSES), jnp.float32),
                   jax.ShapeDtypeStruct((N_PAD, 1), jnp.int32)],
    )
    return call(t2, s2, s2, b2h.reshape(1, HID), Wc,
                bc.reshape(1, NUM_CLASSES), lab_pad)


def kernel(node_subgraph, adj_row, adj_col, adj_val, feat_full, label_full,
           W1_self, b1_self, W1_hop, b1_hop, W2_self, b2_self, W2_hop, b2_hop,
           Wc, bc):
    idx_pad = jnp.pad(node_subgraph, (0, N_PAD - N_SUB))
    lab_full_pad = jnp.pad(label_full, ((0, 0), (0, LAB_PAD - NUM_CLASSES)))
    ep = E_PAD - E
    col2d = jnp.pad(adj_col, (0, ep)).reshape(E_PAD // CHUNK, CHUNK)
    row2d = jnp.pad(adj_row, (0, ep),
                    constant_values=N_PAD - 1).reshape(E_PAD // CHUNK, CHUNK)
    val2d = jnp.pad(adj_val, (0, ep)).reshape(E_PAD // CHUNK, CHUNK)

    feat_pad, lab_pad = _sc_gather(idx_pad, feat_full, lab_full_pad)
    t1, u1p = _tc_layer1(feat_pad, W1_self, W1_hop, b1_self)
    s1 = _sc_spmm(u1p, col2d, row2d, val2d).reshape(2 * N_PAD, 2 * 128)
    t2, u2p = _tc_layer2(t1, s1, b1_hop, W2_self, W2_hop, b2_self)
    s2 = _sc_spmm(u2p, col2d, row2d, val2d).reshape(2 * N_PAD, 2 * 128)
    pred_pad, conv_pad = _tc_final(t2, s2, b2_hop, Wc, bc, lab_pad)

    return (pred_pad[:N_SUB],
            lab_pad[:N_SUB, :NUM_CLASSES],
            conv_pad[:N_SUB, 0])


# speedup vs baseline: 1.8282x; 1.8282x over previous
import functools

import jax
import jax.numpy as jnp
from jax import lax
from jax.experimental import pallas as pl
from jax.experimental.pallas import tpu as pltpu
from jax.experimental.pallas import tpu_sc as plsc

N_FULL = 100000
N_SUB = 10000
E = 160000
F_IN = 256
HID = 256
NUM_CLASSES = 41

NC = 2
NS = 16
N_PAD = 10240
ROWS_W = N_PAD // (NC * NS)
LAB_PAD = 128

CHUNK = 128
NCHUNK = 80
E_PAD = NC * NS * CHUNK * NCHUNK // 2
HALF = N_PAD // 2
ACC_R = HALF + 8
ROWS_P = HALF // NS

_BCAST_DNUMS = lax.GatherDimensionNumbers(
    offset_dims=(), collapsed_slice_dims=(0,), start_index_map=(0,))


def _lane_bcast(v16, lane):
    idx = jnp.full((16, 1), lane, jnp.int32)
    return lax.gather(v16, idx, _BCAST_DNUMS, (1,),
                      mode=lax.GatherScatterMode.PROMISE_IN_BOUNDS)

_mesh = functools.partial(
    plsc.VectorSubcoreMesh, core_axis_name="c", subcore_axis_name="s",
    num_cores=NC, num_subcores=NS)


def _gather_body(idx_hbm, feat_hbm, lab_hbm, feat_out, lab_out,
                 idx_v, feat_v, lab_v, sem):
    wid = lax.axis_index("s") * NC + lax.axis_index("c")
    base = wid * ROWS_W
    pltpu.sync_copy(idx_hbm.at[pl.ds(base, ROWS_W)], idx_v)
    pltpu.async_copy(feat_hbm.at[idx_v], feat_v, sem).wait()
    pltpu.sync_copy(feat_v, feat_out.at[pl.ds(base, ROWS_W)])
    pltpu.async_copy(lab_hbm.at[idx_v], lab_v, sem).wait()
    pltpu.sync_copy(lab_v, lab_out.at[pl.ds(base, ROWS_W)])


def _sc_gather(idx_pad, feat_full, lab_pad_full):
    return pl.kernel(
        _gather_body,
        out_type=(jax.ShapeDtypeStruct((N_PAD, F_IN), jnp.float32),
                  jax.ShapeDtypeStruct((N_PAD, LAB_PAD), jnp.float32)),
        mesh=_mesh(),
        scratch_types=(
            pltpu.VMEM((ROWS_W,), jnp.int32),
            pltpu.VMEM((ROWS_W, F_IN), jnp.float32),
            pltpu.VMEM((ROWS_W, LAB_PAD), jnp.float32),
            pltpu.SemaphoreType.DMA,
        ),
    )(idx_pad, feat_full, lab_pad_full)


def _spmm_body(u_hbm, col_hbm, row_hbm, val_hbm, out_hbm,
               col_v, row_v, val_v, row_adj, gath0, gath1, zero_v, acc,
               g0, g1, s0, s1):
    cid = lax.axis_index("c")
    sid = lax.axis_index("s")

    pltpu.sync_copy(col_hbm.at[pl.ds(sid * NCHUNK, NCHUNK)],
                    col_v.at[pl.ds(0, NCHUNK)])
    pltpu.sync_copy(row_hbm.at[pl.ds(sid * NCHUNK, NCHUNK)],
                    row_v.at[pl.ds(0, NCHUNK)])
    pltpu.sync_copy(val_hbm.at[pl.ds(sid * NCHUNK, NCHUNK)],
                    val_v.at[pl.ds(0, NCHUNK)])
    zv16 = jnp.zeros((16,), jnp.float32)
    for j in range(8):
        col_v[NCHUNK, pl.ds(j * 16, 16)] = jnp.zeros((16,), jnp.int32)
        row_v[NCHUNK, pl.ds(j * 16, 16)] = jnp.zeros((16,), jnp.int32)
        val_v[NCHUNK, pl.ds(j * 16, 16)] = zv16

    u_half = u_hbm.at[pl.ds(cid * N_PAD, N_PAD)]

    def _zrow(i, _):
        for j in range(8):
            zero_v[i, pl.ds(j * 16, 16)] = zv16
        return 0
    lax.fori_loop(0, 64, _zrow, 0)

    def _scale(gb, k):
        def _grp(b, _):
            val16 = val_v[k, pl.ds(pl.multiple_of(b * 16, 16), 16)]
            for lane in range(16):
                vb = _lane_bcast(val16, lane)
                j = b * 16 + lane
                for f in range(8):
                    gb[j, pl.ds(f * 16, 16)] = gb[j, pl.ds(f * 16, 16)] * vb
            return 0
        lax.fori_loop(0, CHUNK // 16, _grp, 0)

    for p in range(2):
        lo = p * HALF
        for z in range(5):
            pltpu.sync_copy(zero_v, acc.at[pl.ds(sid * ROWS_P + z * 64, 64)])
        plsc.subcore_barrier()

        def _radj(i, _):
            for j in range(8):
                r = row_v[i, pl.ds(j * 16, 16)] - lo
                ok = (r >= 0) & (r < HALF)
                row_adj[i, pl.ds(j * 16, 16)] = jnp.where(ok, r, HALF)
            return 0
        lax.fori_loop(0, NCHUNK + 1, _radj, 0)

        def _scan(k, carry):
            below, above = carry
            cmax = row_v[k, pl.ds(CHUNK - 16, 16)][15]
            cmin = row_v[k, pl.ds(0, 16)][0]
            below = below + jnp.where(cmax < lo, 1, 0)
            above = above + jnp.where(cmin >= lo + HALF, 1, 0)
            return (below, above)
        below, above = lax.fori_loop(0, NCHUNK, _scan, (0, 0))
        k_lo = below
        k_hi = NCHUNK - above

        ka = jnp.minimum(k_lo, NCHUNK)
        kb = jnp.minimum(k_lo + 1, NCHUNK)
        pltpu.async_copy(u_half.at[col_v.at[ka]], gath0, g0)
        pltpu.async_copy(u_half.at[col_v.at[kb]], gath1, g1)
        npairs = (k_hi - k_lo + 1) // 2

        def _pair(i, _):
            for h, gb, gs, ss in ((0, gath0, g0, s0), (1, gath1, g1, s1)):
                k = k_lo + 2 * i + h
                pltpu.make_async_copy(u_half.at[col_v.at[k]], gb, gs).wait()
                _scale(gb, k)
                pltpu.async_copy(gb, acc.at[row_adj.at[k]], ss, add=True)
                pltpu.make_async_copy(gb, acc.at[row_adj.at[k]], ss).wait()
                kp = jnp.minimum(k + 2, NCHUNK)
                pltpu.async_copy(u_half.at[col_v.at[kp]], gb, gs)
            return 0
        lax.fori_loop(0, npairs, _pair, 0)

        pltpu.make_async_copy(u_half.at[col_v.at[0]], gath0, g0).wait()
        pltpu.make_async_copy(u_half.at[col_v.at[0]], gath1, g1).wait()

        plsc.subcore_barrier()
        pltpu.sync_copy(
            acc.at[pl.ds(sid * ROWS_P, ROWS_P)],
            out_hbm.at[pl.ds(cid * N_PAD + lo + sid * ROWS_P, ROWS_P)])
        plsc.subcore_barrier()


def _sc_spmm(u_flat, col2d, row2d, val2d):
    return pl.kernel(
        _spmm_body,
        out_type=jax.ShapeDtypeStruct((2 * N_PAD, 128), jnp.float32),
        name="spmm",
        mesh=_mesh(),
        scratch_types=(
            pltpu.VMEM((NCHUNK + 1, CHUNK), jnp.int32),
            pltpu.VMEM((NCHUNK + 1, CHUNK), jnp.int32),
            pltpu.VMEM((NCHUNK + 1, CHUNK), jnp.float32),
            pltpu.VMEM((NCHUNK + 1, CHUNK), jnp.int32),
            pltpu.VMEM((CHUNK, 128), jnp.float32),
            pltpu.VMEM((CHUNK, 128), jnp.float32),
            pltpu.VMEM((64, 128), jnp.float32),
            pltpu.VMEM_SHARED((ACC_R, 128), jnp.float32),
            pltpu.SemaphoreType.DMA,
            pltpu.SemaphoreType.DMA,
            pltpu.SemaphoreType.DMA,
            pltpu.SemaphoreType.DMA,
        ),
    )(u_flat, col2d, row2d, val2d)


def _k1_body(x_ref, ws_ref, wh_ref, bs_ref, t1_ref, u_ref):
    x = x_ref[...]
    t1_ref[...] = jnp.maximum(
        jnp.dot(x, ws_ref[...], preferred_element_type=jnp.float32)
        + bs_ref[...], 0.0)
    u_ref[...] = jnp.dot(x, wh_ref[...], preferred_element_type=jnp.float32)


def _tc_layer1(feat, W1s, W1h, b1s):
    nb = N_PAD // 1024
    return pl.pallas_call(
        _k1_body,
        grid=(nb, 2),
        in_specs=[
            pl.BlockSpec((1024, F_IN), lambda i, j: (i, 0)),
            pl.BlockSpec((F_IN, 128), lambda i, j: (0, j)),
            pl.BlockSpec((F_IN, 128), lambda i, j: (0, j)),
            pl.BlockSpec((1, 128), lambda i, j: (0, j)),
        ],
        out_specs=[
            pl.BlockSpec((1024, 128), lambda i, j: (i, j)),
            pl.BlockSpec((1024, 128), lambda i, j: (j * nb + i, 0)),
        ],
        out_shape=[jax.ShapeDtypeStruct((N_PAD, HID), jnp.float32),
                   jax.ShapeDtypeStruct((2 * N_PAD, 128), jnp.float32)],
    )(feat, W1s, W1h, b1s.reshape(1, HID))


def _k2_body(t1_ref, s1a_ref, s1b_ref, b1a_ref, b1b_ref,
             w2s_ref, w2h_ref, b2s_ref, t2_ref, u2_ref):
    t1 = t1_ref[...]
    p1a = jnp.maximum(s1a_ref[...] + b1a_ref[...], 0.0)
    p1b = jnp.maximum(s1b_ref[...] + b1b_ref[...], 0.0)
    w2s = w2s_ref[...]
    w2h = w2h_ref[...]
    acc = (jnp.dot(t1, w2s[:HID], preferred_element_type=jnp.float32)
           + jnp.dot(p1a, w2s[HID:HID + 128], preferred_element_type=jnp.float32)
           + jnp.dot(p1b, w2s[HID + 128:], preferred_element_type=jnp.float32))
    t2_ref[...] = jnp.maximum(acc + b2s_ref[...], 0.0)
    u2_ref[...] = (jnp.dot(t1, w2h[:HID], preferred_element_type=jnp.float32)
                   + jnp.dot(p1a, w2h[HID:HID + 128], preferred_element_type=jnp.float32)
                   + jnp.dot(p1b, w2h[HID + 128:], preferred_element_type=jnp.float32))


def _tc_layer2(t1, s1, b1h, W2s, W2h, b2s):
    nb = N_PAD // 1024
    call = pl.pallas_call(
        _k2_body,
        grid=(nb, 2),
        in_specs=[
            pl.BlockSpec((1024, HID), lambda i, j: (i, 0)),
            pl.BlockSpec((1024, 128), lambda i, j: (i, 0)),
            pl.BlockSpec((1024, 128), lambda i, j: (nb + i, 0)),
            pl.BlockSpec((1, 128), lambda i, j: (0, 0)),
            pl.BlockSpec((1, 128), lambda i, j: (0, 1)),
            pl.BlockSpec((2 * HID, 128), lambda i, j: (0, j)),
            pl.BlockSpec((2 * HID, 128), lambda i, j: (0, j)),
            pl.BlockSpec((1, 128), lambda i, j: (0, j)),
        ],
        out_specs=[
            pl.BlockSpec((1024, 128), lambda i, j: (i, j)),
            pl.BlockSpec((1024, 128), lambda i, j: (j * nb + i, 0)),
        ],
        out_shape=[jax.ShapeDtypeStruct((N_PAD, HID), jnp.float32),
                   jax.ShapeDtypeStruct((2 * N_PAD, 128), jnp.float32)],
    )
    b1h2 = b1h.reshape(1, HID)
    return call(t1, s1, s1, b1h2, b1h2, W2s, W2h, b2s.reshape(1, HID))


def _k3_body(t2_ref, s2a_ref, s2b_ref, b2a_ref, b2b_ref, wc_ref, bc_ref,
             lab_ref, pred_ref, conv_ref):
    t2 = t2_ref[...]
    p2a = jnp.maximum(s2a_ref[...] + b2a_ref[...], 0.0)
    p2b = jnp.maximum(s2b_ref[...] + b2b_ref[...], 0.0)
    wc = wc_ref[...]
    z = (jnp.dot(t2, wc[:HID], preferred_element_type=jnp.float32)
         + jnp.dot(p2a, wc[HID:HID + 128], preferred_element_type=jnp.float32)
         + jnp.dot(p2b, wc[HID + 128:], preferred_element_type=jnp.float32))
    nsq = (jnp.sum(t2 * t2, axis=1, keepdims=True)
           + jnp.sum(p2a * p2a, axis=1, keepdims=True)
           + jnp.sum(p2b * p2b, axis=1, keepdims=True))
    n = jnp.maximum(jnp.sqrt(nsq), 1e-12)
    pred_ref[...] = z / n + bc_ref[...]
    lab = lab_ref[...][:, :NUM_CLASSES]
    m = jnp.max(lab, axis=1, keepdims=True)
    ii = lax.broadcasted_iota(jnp.int32, lab.shape, 1)
    conv_ref[...] = jnp.min(
        jnp.where(lab == m, ii, NUM_CLASSES), axis=1, keepdims=True)


def _tc_final(t2, s2, b2h, Wc, bc, lab_pad):
    nb = N_PAD // 1024
    call = pl.pallas_call(
        _k3_body,
        grid=(nb,),
        in_specs=[
            pl.BlockSpec((1024, HID), lambda i: (i, 0)),
            pl.BlockSpec((1024, 128), lambda i: (i, 0)),
            pl.BlockSpec((1024, 128), lambda i: (nb + i, 0)),
            pl.BlockSpec((1, 128), lambda i: (0, 0)),
            pl.BlockSpec((1, 128), lambda i: (0, 1)),
            pl.BlockSpec((2 * HID, NUM_CLASSES), lambda i: (0, 0)),
            pl.BlockSpec((1, NUM_CLASSES), lambda i: (0, 0)),
            pl.BlockSpec((1024, LAB_PAD), lambda i: (i, 0)),
        ],
        out_specs=[
            pl.BlockSpec((1024, NUM_CLASSES), lambda i: (i, 0)),
            pl.BlockSpec((1024, 1), lambda i: (i, 0)),
        ],
        out_shape=[jax.ShapeDtypeStruct((N_PAD, NUM_CLASSES), jnp.float32),
                   jax.ShapeDtypeStruct((N_PAD, 1), jnp.int32)],
    )
    b2h2 = b2h.reshape(1, HID)
    return call(t2, s2, s2, b2h2, b2h2, Wc, bc.reshape(1, NUM_CLASSES), lab_pad)


def kernel(node_subgraph, adj_row, adj_col, adj_val, feat_full, label_full,
           W1_self, b1_self, W1_hop, b1_hop, W2_self, b2_self, W2_hop, b2_hop,
           Wc, bc):
    idx_pad = jnp.pad(node_subgraph, (0, N_PAD - N_SUB))
    lab_full_pad = jnp.pad(label_full, ((0, 0), (0, LAB_PAD - NUM_CLASSES)))
    ep = E_PAD - E
    col2d = jnp.pad(adj_col, (0, ep)).reshape(NS * NCHUNK, CHUNK)
    row2d = jnp.pad(adj_row, (0, ep),
                    constant_values=N_PAD - 1).reshape(NS * NCHUNK, CHUNK)
    val2d = jnp.pad(adj_val, (0, ep)).reshape(NS * NCHUNK, CHUNK)

    feat_pad, lab_pad = _sc_gather(idx_pad, feat_full, lab_full_pad)
    t1, u1 = _tc_layer1(feat_pad, W1_self, W1_hop, b1_self)
    s1 = _sc_spmm(u1, col2d, row2d, val2d)
    t2, u2 = _tc_layer2(t1, s1, b1_hop, W2_self, W2_hop, b2_self)
    s2 = _sc_spmm(u2, col2d, row2d, val2d)
    pred_pad, conv_pad = _tc_final(t2, s2, b2_hop, Wc, bc, lab_pad)

    return (pred_pad[:N_SUB],
            lab_pad[:N_SUB, :NUM_CLASSES],
            conv_pad[:N_SUB, 0])
